# baseline (speedup 1.0000x reference)
import functools

import jax
import jax.numpy as jnp
from jax import lax
from jax.experimental import pallas as pl
from jax.experimental.pallas import tpu as pltpu
from jax.experimental.pallas import tpu_sc as plsc

B = 16384
D = 64
NW = 32
BPW = B // NW
NCHUNK = 4
CW = BPW // NCHUNK


def _sc_gather_body(u_hbm, i_hbm, uidx_hbm, iidx_hbm, hu_hbm, hi_hbm,
                    uidx_v, iidx_v, rows_u, rows_i, sem_u, sem_i):
    wid = lax.axis_index("s") * 2 + lax.axis_index("c")
    base = wid * BPW
    pltpu.sync_copy(uidx_hbm.at[wid], uidx_v)
    pltpu.sync_copy(iidx_hbm.at[wid], iidx_v)
    copies = []
    for c in range(NCHUNK):
        copies.append(pltpu.async_copy(
            u_hbm.at[uidx_v.at[c]], rows_u.at[pl.ds(c * CW, CW)], sem_u))
        copies.append(pltpu.async_copy(
            i_hbm.at[iidx_v.at[c]], rows_i.at[pl.ds(c * CW, CW)], sem_i))
    for cp in copies:
        cp.wait()
    pltpu.sync_copy(rows_u, hu_hbm.at[pl.ds(base, BPW)])
    pltpu.sync_copy(rows_i, hi_hbm.at[pl.ds(base, BPW)])


@jax.jit
def _sc_gather(U, I, uidx, iidx):
    mesh = plsc.VectorSubcoreMesh(core_axis_name="c", subcore_axis_name="s")
    return pl.kernel(
        _sc_gather_body,
        out_type=(
            jax.ShapeDtypeStruct((B, D), jnp.float32),
            jax.ShapeDtypeStruct((B, D), jnp.float32),
        ),
        mesh=mesh,
        compiler_params=pltpu.CompilerParams(use_tc_tiling_on_sc=False),
        scratch_types=[
            pltpu.VMEM((NCHUNK, CW), jnp.int32),
            pltpu.VMEM((NCHUNK, CW), jnp.int32),
            pltpu.VMEM((BPW, D), jnp.float32),
            pltpu.VMEM((BPW, D), jnp.float32),
            pltpu.SemaphoreType.DMA,
            pltpu.SemaphoreType.DMA,
        ],
    )(U, I, uidx, iidx)


def _mlp_body(hu_ref, hi_ref, w1a_ref, w1b_ref, b1_ref, w2_ref, b2_ref, o_ref):
    h = (jnp.dot(hu_ref[...], w1a_ref[...], preferred_element_type=jnp.float32)
         + jnp.dot(hi_ref[...], w1b_ref[...], preferred_element_type=jnp.float32)
         + b1_ref[...])
    h = jnp.maximum(h, 0.0)
    o = jnp.dot(h, w2_ref[...], preferred_element_type=jnp.float32) + b2_ref[...]
    o_ref[...] = jax.nn.sigmoid(o) * 5.0 + 0.5


@functools.partial(jax.jit, static_argnames=("block_b",))
def _mlp(hu, hi, w1a, w1b, b1, w2, b2, block_b=2048):
    nblocks = B // block_b
    return pl.pallas_call(
        _mlp_body,
        grid=(nblocks,),
        in_specs=[
            pl.BlockSpec((block_b, D), lambda i: (i, 0)),
            pl.BlockSpec((block_b, D), lambda i: (i, 0)),
            pl.BlockSpec((D, 10), lambda i: (0, 0)),
            pl.BlockSpec((D, 10), lambda i: (0, 0)),
            pl.BlockSpec((1, 10), lambda i: (0, 0)),
            pl.BlockSpec((10, 1), lambda i: (0, 0)),
            pl.BlockSpec((1, 1), lambda i: (0, 0)),
        ],
        out_specs=pl.BlockSpec((block_b, 1), lambda i: (i, 0)),
        out_shape=jax.ShapeDtypeStruct((B, 1), jnp.float32),
    )(hu, hi, w1a, w1b, b1, w2, b2)


def kernel(x, U, I, W1, b1, W2, b2):
    uidx = x[:, 0].astype(jnp.int32).reshape(NW, NCHUNK, CW)
    iidx = x[:, 1].astype(jnp.int32).reshape(NW, NCHUNK, CW)
    hu, hi = _sc_gather(U, I, uidx, iidx)
    out = _mlp(hu, hi, W1[:D], W1[D:], b1.reshape(1, 10),
               W2, b2.reshape(1, 1))
    return out
